# baseline (speedup 1.0000x reference)
import numpy as np
import jax
import jax.numpy as jnp
from jax.experimental import pallas as pl
from jax.experimental.pallas import tpu as pltpu

_N = 4096
_D = 16
_K = 8
_CUTOFF = 1.4
_RB = 512
_GRID = _N // _RB
_TINY = float(np.finfo(np.float32).tiny)

_ROTS = ((13, 15, 26, 6), (17, 29, 16, 24), (13, 15, 26, 6),
         (17, 29, 16, 24), (13, 15, 26, 6))


def _np_threefry_pair(k1, k2, c0, c1):
    ks = [np.uint32(k1), np.uint32(k2),
          np.uint32(np.uint32(k1) ^ np.uint32(k2) ^ np.uint32(0x1BD11BDA))]
    x0 = int(np.uint32(c0) + ks[0]) % 2**32
    x1 = (int(c1) + int(ks[1])) % 2**32
    for i in range(5):
        for r in _ROTS[i]:
            x0 = (x0 + x1) % 2**32
            x1 = ((x1 << r) | (x1 >> (32 - r))) % 2**32
            x1 ^= x0
        x0 = (x0 + int(ks[(i + 1) % 3])) % 2**32
        x1 = (x1 + int(ks[(i + 2) % 3]) + i + 1) % 2**32
    return x0, x1


_K1, _K2 = _np_threefry_pair(0, 0, 0, 123)


def _threefry_xor(x1):
    k1 = jnp.uint32(_K1)
    k2 = jnp.uint32(_K2)
    ks = (k1, k2, jnp.uint32(_K1 ^ _K2 ^ 0x1BD11BDA))
    x0 = jnp.full(x1.shape, k1, jnp.uint32)
    x1 = x1 + k2
    for i in range(5):
        for r in _ROTS[i]:
            x0 = x0 + x1
            x1 = (x1 << jnp.uint32(r)) | (x1 >> jnp.uint32(32 - r))
            x1 = x1 ^ x0
        x0 = x0 + ks[(i + 1) % 3]
        x1 = x1 + ks[(i + 2) % 3] + jnp.uint32(i + 1)
    return x0 ^ x1


def _logw_block(xt, xb, b):
    f = jnp.float32
    nrmsq = jnp.sum(xt * xt, axis=0, keepdims=True)
    xnt = xt * (f(1.0) / (jnp.sqrt(nrmsq) + f(1e-5)))
    sq_all = jnp.sum(xnt * xnt, axis=0, keepdims=True)
    r_b = jnp.sum(xb * xb, axis=1, keepdims=True)
    xnb = xb * (f(1.0) / (jnp.sqrt(r_b) + f(1e-5)))
    sq_b = jnp.sum(xnb * xnb, axis=1, keepdims=True)
    prod = jax.lax.dot_general(
        xnb, xnt, (((1,), (0,)), ((), ())),
        precision=jax.lax.Precision.HIGHEST,
        preferred_element_type=jnp.float32)
    dist2 = sq_b + sq_all - f(2.0) * prod
    ig = b * _RB + jax.lax.broadcasted_iota(jnp.int32, (_RB, 1), 0)
    col = jax.lax.broadcasted_iota(jnp.int32, (_RB, _N), 1)
    eye = (col == ig).astype(jnp.float32)
    d2c = jnp.maximum(dist2 + eye, f(1e-12))
    logw = (f((2.0 - _D) / 2.0) * jnp.log(d2c)
            - f((_D - 3.0) / 2.0)
            * jnp.log(jnp.maximum(f(1.0) - f(0.25) * d2c, f(1e-12))))
    return d2c, logw, ig, col, nrmsq


def _main_kernel(xt_ref, xb_ref, out_ref):
    f = jnp.float32
    b = pl.program_id(0)
    xt = xt_ref[...]
    xb = xb_ref[...]
    d2c, logw, ig, col, nrmsq = _logw_block(xt, xb, b)

    row_m = jnp.max(logw, axis=1, keepdims=True)
    w = jnp.exp(logw - row_m)
    mask = ((col >> 3) != (ig >> 3)) & (d2c < f(_CUTOFF * _CUTOFF))
    w = jnp.where(mask, w, f(0.0))
    wsum = jnp.sum(w, axis=1, keepdims=True)
    probs = jnp.where(wsum > f(0.0),
                      w / jnp.where(wsum > f(0.0), wsum, f(1.0)),
                      f(1.0 / _N))
    logits = jnp.log(probs + f(1e-30))

    imod = ig & 7
    il = jax.lax.broadcasted_iota(jnp.int32, (_RB, 1), 0)
    lcol = jax.lax.broadcasted_iota(jnp.int32, (_RB, _RB), 1)
    colu = col.astype(jnp.uint32)
    igu = ig.astype(jnp.uint32)

    def t_body(t, total):
        tu = t.astype(jnp.uint32)
        flat = (tu * jnp.uint32(_N) + igu) * jnp.uint32(_N) + colu
        bits = _threefry_xor(flat)
        m23 = (bits >> jnp.uint32(9)).astype(jnp.int32).astype(jnp.float32)
        u = jnp.maximum(m23 * f(2.0 ** -23), f(_TINY))
        val = logits - jnp.log(-jnp.log(u))
        neg = jnp.argmax(val, axis=1).astype(jnp.int32)[:, None]
        ohn = (col == neg).astype(jnp.float32)
        xneg = jax.lax.dot_general(
            ohn, xt, (((1,), (1,)), ((), ())),
            precision=jax.lax.Precision.HIGHEST,
            preferred_element_type=jnp.float32)
        dn = jnp.sqrt(jnp.sum((xb - xneg + f(1e-6)) ** 2,
                              axis=1, keepdims=True))
        p_loc = ((il >> 3) << 3) + t + (t >= imod).astype(jnp.int32)
        ohp = (lcol == p_loc).astype(jnp.float32)
        xp = jax.lax.dot_general(
            ohp, xb, (((1,), (0,)), ((), ())),
            precision=jax.lax.Precision.HIGHEST,
            preferred_element_type=jnp.float32)
        dp = jnp.sqrt(jnp.sum((xb - xp + f(1e-6)) ** 2,
                              axis=1, keepdims=True))
        return total + jnp.sum(jnp.maximum(dp - dn + f(1.0), f(0.0)))

    total = jax.lax.fori_loop(0, _K - 1, t_body, f(0.0))
    out_ref[...] = jnp.full((1, 1, 128), total, jnp.float32)


def kernel(x):
    xt = x.T
    parts = pl.pallas_call(
        _main_kernel,
        grid=(_GRID,),
        in_specs=[
            pl.BlockSpec((_D, _N), lambda b: (0, 0)),
            pl.BlockSpec((_RB, _D), lambda b: (b, 0)),
        ],
        out_specs=pl.BlockSpec((1, 1, 128), lambda b: (b, 0, 0)),
        out_shape=jax.ShapeDtypeStruct((_GRID, 1, 128), jnp.float32),
        compiler_params=pltpu.CompilerParams(
            dimension_semantics=("parallel",)),
    )(xt, x)
    return (jnp.sum(parts[:, 0, 0]) / jnp.float32(_N * (_K - 1))).astype(jnp.float32)


# speedup vs baseline: 1.1968x; 1.1968x over previous
import numpy as np
import jax
import jax.numpy as jnp
from jax.experimental import pallas as pl
from jax.experimental.pallas import tpu as pltpu

_N = 4096
_D = 16
_K = 8
_CUTOFF = 1.4
_RB = 512
_GRID = _N // _RB
_TINY = float(np.finfo(np.float32).tiny)

_ROTS = ((13, 15, 26, 6), (17, 29, 16, 24), (13, 15, 26, 6),
         (17, 29, 16, 24), (13, 15, 26, 6))


def _np_threefry_pair(k1, k2, c0, c1):
    ks = [np.uint32(k1), np.uint32(k2),
          np.uint32(np.uint32(k1) ^ np.uint32(k2) ^ np.uint32(0x1BD11BDA))]
    x0 = int(np.uint32(c0) + ks[0]) % 2**32
    x1 = (int(c1) + int(ks[1])) % 2**32
    for i in range(5):
        for r in _ROTS[i]:
            x0 = (x0 + x1) % 2**32
            x1 = ((x1 << r) | (x1 >> (32 - r))) % 2**32
            x1 ^= x0
        x0 = (x0 + int(ks[(i + 1) % 3])) % 2**32
        x1 = (x1 + int(ks[(i + 2) % 3]) + i + 1) % 2**32
    return x0, x1


_K1, _K2 = _np_threefry_pair(0, 0, 0, 123)


def _threefry_xor(x1):
    k1 = jnp.uint32(_K1)
    k2 = jnp.uint32(_K2)
    ks = (k1, k2, jnp.uint32(_K1 ^ _K2 ^ 0x1BD11BDA))
    x0 = jnp.full(x1.shape, k1, jnp.uint32)
    x1 = x1 + k2
    for i in range(5):
        for r in _ROTS[i]:
            x0 = x0 + x1
            x1 = (x1 << jnp.uint32(r)) | (x1 >> jnp.uint32(32 - r))
            x1 = x1 ^ x0
        x0 = x0 + ks[(i + 1) % 3]
        x1 = x1 + ks[(i + 2) % 3] + jnp.uint32(i + 1)
    return x0 ^ x1


def _logw_block(xt, xb, b):
    f = jnp.float32
    nrmsq = jnp.sum(xt * xt, axis=0, keepdims=True)
    xnt = xt * (f(1.0) / (jnp.sqrt(nrmsq) + f(1e-5)))
    sq_all = jnp.sum(xnt * xnt, axis=0, keepdims=True)
    r_b = jnp.sum(xb * xb, axis=1, keepdims=True)
    xnb = xb * (f(1.0) / (jnp.sqrt(r_b) + f(1e-5)))
    sq_b = jnp.sum(xnb * xnb, axis=1, keepdims=True)
    prod = jax.lax.dot_general(
        xnb, xnt, (((1,), (0,)), ((), ())),
        precision=jax.lax.Precision.HIGHEST,
        preferred_element_type=jnp.float32)
    dist2 = sq_b + sq_all - f(2.0) * prod
    ig = b * _RB + jax.lax.broadcasted_iota(jnp.int32, (_RB, 1), 0)
    col = jax.lax.broadcasted_iota(jnp.int32, (_RB, _N), 1)
    eye = (col == ig).astype(jnp.float32)
    d2c = jnp.maximum(dist2 + eye, f(1e-12))
    logw = (f((2.0 - _D) / 2.0) * jnp.log(d2c)
            - f((_D - 3.0) / 2.0)
            * jnp.log(jnp.maximum(f(1.0) - f(0.25) * d2c, f(1e-12))))
    return d2c, logw, ig, col, nrmsq


def _main_kernel(xt_ref, xb_ref, out_ref):
    f = jnp.float32
    b = pl.program_id(0)
    xt = xt_ref[...]
    xb = xb_ref[...]
    d2c, logw, ig, col, nrmsq = _logw_block(xt, xb, b)

    row_m = jnp.max(logw, axis=1, keepdims=True)
    w = jnp.exp(logw - row_m)
    mask = ((col >> 3) != (ig >> 3)) & (d2c < f(_CUTOFF * _CUTOFF))
    w = jnp.where(mask, w, f(0.0))
    wsum = jnp.sum(w, axis=1, keepdims=True)
    probs = jnp.where(wsum > f(0.0),
                      w / jnp.where(wsum > f(0.0), wsum, f(1.0)),
                      f(1.0 / _N))
    logits = jnp.log(probs + f(1e-30))

    s_all = jnp.sum(xt, axis=0, keepdims=True)
    s_b = jnp.sum(xb, axis=1, keepdims=True)
    r_b = jnp.sum(xb * xb, axis=1, keepdims=True)
    graw = jax.lax.dot_general(
        xb, xt, (((1,), (0,)), ((), ())),
        precision=jax.lax.Precision.HIGHEST,
        preferred_element_type=jnp.float32)
    payload = nrmsq - f(2e-6) * s_all - f(2.0) * graw
    base = r_b + f(_D * 1e-12) + f(2e-6) * s_b

    imod = ig & 7
    iblk = ig >> 3
    colu = col.astype(jnp.uint32)
    igu = ig.astype(jnp.uint32)

    def t_body(t, total):
        tu = t.astype(jnp.uint32)
        flat = (tu * jnp.uint32(_N) + igu) * jnp.uint32(_N) + colu
        bits = _threefry_xor(flat)
        m23 = (bits >> jnp.uint32(9)).astype(jnp.int32).astype(jnp.float32)
        u = jnp.maximum(m23 * f(2.0 ** -23), f(_TINY))
        val = logits - jnp.log(-jnp.log(u))
        neg = jnp.argmax(val, axis=1).astype(jnp.int32)[:, None]
        ohn = (col == neg).astype(jnp.float32)
        dn = jnp.sqrt(jnp.maximum(
            base + jnp.sum(ohn * payload, axis=1, keepdims=True), f(0.0)))
        p = (iblk << 3) + t + (t >= imod).astype(jnp.int32)
        ohp = (col == p).astype(jnp.float32)
        dp = jnp.sqrt(jnp.maximum(
            base + jnp.sum(ohp * payload, axis=1, keepdims=True), f(0.0)))
        return total + jnp.sum(jnp.maximum(dp - dn + f(1.0), f(0.0)))

    total = jax.lax.fori_loop(0, _K - 1, t_body, f(0.0))
    out_ref[...] = jnp.full((1, 1, 128), total, jnp.float32)


def kernel(x):
    xt = x.T
    parts = pl.pallas_call(
        _main_kernel,
        grid=(_GRID,),
        in_specs=[
            pl.BlockSpec((_D, _N), lambda b: (0, 0)),
            pl.BlockSpec((_RB, _D), lambda b: (b, 0)),
        ],
        out_specs=pl.BlockSpec((1, 1, 128), lambda b: (b, 0, 0)),
        out_shape=jax.ShapeDtypeStruct((_GRID, 1, 128), jnp.float32),
        compiler_params=pltpu.CompilerParams(
            dimension_semantics=("parallel",)),
    )(xt, x)
    return (jnp.sum(parts[:, 0, 0]) / jnp.float32(_N * (_K - 1))).astype(jnp.float32)
